# Initial kernel scaffold; baseline (speedup 1.0000x reference)
import jax
import jax.numpy as jnp
from jax.experimental import pallas as pl


def kernel(x, edge_index, edge_attr, Wq, bq, Wk, bk, Wv, bv, We, be, Wo, bo, g1, bn1, g2, bn2, W1, bf1, W2, bf2):
    raise NotImplementedError("write your pallas kernel here")


# speedup vs baseline: 39.0507x; 39.0507x over previous
import functools

import jax
import jax.numpy as jnp
from jax import lax
from jax.experimental import pallas as pl
from jax.experimental.pallas import tpu as pltpu
from jax.experimental.pallas import tpu_sc as plsc

N = 10000
E = 320000
HD = 128
ED = 16
H = 8
D = 16
FF = 4 * HD

NC = 2
NS = 16
NW = NC * NS
EPT = E // NW
C = 400
SUB = 50
NSUB = C // SUB
NCH = EPT // C
NPAD = 10240
RPS = NPAD // NS
EB = 4000
SCALE = 1.0 / (D ** 0.5)

_f32 = jnp.float32


def _qkv_body(x_ref, wq_ref, bq_ref, wk_ref, bk_ref, wv_ref, bv_ref,
              q_ref, k_ref, v_ref):
    xb = x_ref[...]
    q_ref[...] = jnp.dot(xb, wq_ref[...], preferred_element_type=_f32) + bq_ref[...]
    k_ref[...] = jnp.dot(xb, wk_ref[...], preferred_element_type=_f32) + bk_ref[...]
    v_ref[...] = jnp.dot(xb, wv_ref[...], preferred_element_type=_f32) + bv_ref[...]


def _qkv_call(x, Wq, bq, Wk, bk, Wv, bv):
    blk = 2000
    grid = (N // blk,)
    full = lambda shape: pl.BlockSpec(shape, lambda i: (0, 0))
    rows = pl.BlockSpec((blk, HD), lambda i: (i, 0))
    return pl.pallas_call(
        _qkv_body,
        grid=grid,
        in_specs=[rows, full((HD, HD)), full((1, HD)), full((HD, HD)),
                  full((1, HD)), full((HD, HD)), full((1, HD))],
        out_specs=[rows, rows, rows],
        out_shape=[jax.ShapeDtypeStruct((N, HD), _f32)] * 3,
    )(x, Wq, bq, Wk, bk, Wv, bv)


def _score_body(qg_ref, kg_ref, ea_ref, we_ref, be_ref, s_ref, m_ref, z_ref):
    i = pl.program_id(0)
    p = qg_ref[...] * kg_ref[...]
    row = lax.broadcasted_iota(jnp.int32, (HD, H), 0)
    col = lax.broadcasted_iota(jnp.int32, (HD, H), 1)
    maskm = (row // D == col).astype(_f32)
    s = (jnp.dot(p, maskm, preferred_element_type=_f32) * SCALE
         + jnp.dot(ea_ref[...], we_ref[...], preferred_element_type=_f32)
         + be_ref[...])
    s_ref[...] = s
    bm = jnp.max(s, axis=0, keepdims=True)

    @pl.when(i == 0)
    def _():
        m_ref[...] = bm
        z_ref[...] = jnp.sum(jnp.exp(s - bm), axis=0, keepdims=True)

    @pl.when(i != 0)
    def _():
        mo = m_ref[...]
        mn = jnp.maximum(mo, bm)
        z_ref[...] = (z_ref[...] * jnp.exp(mo - mn)
                      + jnp.sum(jnp.exp(s - mn), axis=0, keepdims=True))
        m_ref[...] = mn


def _score_call(qg, kg, edge_attr, We, be):
    grid = (E // EB,)
    rows = pl.BlockSpec((EB, HD), lambda i: (i, 0))
    return pl.pallas_call(
        _score_body,
        grid=grid,
        in_specs=[rows, rows, pl.BlockSpec((EB, ED), lambda i: (i, 0)),
                  pl.BlockSpec((ED, H), lambda i: (0, 0)),
                  pl.BlockSpec((1, H), lambda i: (0, 0))],
        out_specs=[pl.BlockSpec((EB, H), lambda i: (i, 0)),
                   pl.BlockSpec((1, H), lambda i: (0, 0)),
                   pl.BlockSpec((1, H), lambda i: (0, 0))],
        out_shape=[jax.ShapeDtypeStruct((E, H), _f32),
                   jax.ShapeDtypeStruct((1, H), _f32),
                   jax.ShapeDtypeStruct((1, H), _f32)],
    )(qg, kg, edge_attr, We, be)


def _wv_body(s_ref, m_ref, z_ref, vg_ref, o_ref):
    w = jnp.exp(s_ref[...] - m_ref[...]) / z_ref[...]
    row = lax.broadcasted_iota(jnp.int32, (H, HD), 0)
    col = lax.broadcasted_iota(jnp.int32, (H, HD), 1)
    expm = (col // D == row).astype(_f32)
    o_ref[...] = jnp.dot(w, expm, preferred_element_type=_f32) * vg_ref[...]


def _wv_call(s, m, z, vg):
    grid = (E // EB,)
    return pl.pallas_call(
        _wv_body,
        grid=grid,
        in_specs=[pl.BlockSpec((EB, H), lambda i: (i, 0)),
                  pl.BlockSpec((1, H), lambda i: (0, 0)),
                  pl.BlockSpec((1, H), lambda i: (0, 0)),
                  pl.BlockSpec((EB, HD), lambda i: (i, 0))],
        out_specs=pl.BlockSpec((EB, HD), lambda i: (i, 0)),
        out_shape=jax.ShapeDtypeStruct((E, HD), _f32),
    )(s, m, z, vg)


def _ln(t, g, b):
    m = jnp.mean(t, axis=-1, keepdims=True)
    c = t - m
    v = jnp.mean(c * c, axis=-1, keepdims=True)
    return c * jax.lax.rsqrt(v + 1e-5) * g + b


def _post_body(x_ref, p_ref, wo_ref, bo_ref, g1_ref, b1_ref, g2_ref, b2_ref,
               w1_ref, bf1_ref, w2_ref, bf2_ref, o_ref):
    att = p_ref[0] + p_ref[1]
    y = jnp.dot(att, wo_ref[...], preferred_element_type=_f32) + bo_ref[...]
    x1 = _ln(x_ref[...] + y, g1_ref[...], b1_ref[...])
    h1 = jnp.maximum(jnp.dot(x1, w1_ref[...], preferred_element_type=_f32)
                     + bf1_ref[...], 0.0)
    ffo = jnp.dot(h1, w2_ref[...], preferred_element_type=_f32) + bf2_ref[...]
    o_ref[...] = _ln(x1 + ffo, g2_ref[...], b2_ref[...])


def _post_call(x, parts, Wo, bo, g1, bn1, g2, bn2, W1, bf1, W2, bf2):
    blk = 2000
    grid = (N // blk,)
    rows = pl.BlockSpec((blk, HD), lambda i: (i, 0))
    full = lambda shape: pl.BlockSpec(shape, lambda i: tuple(0 for _ in shape))
    return pl.pallas_call(
        _post_body,
        grid=grid,
        in_specs=[rows, pl.BlockSpec((NC, blk, HD), lambda i: (0, i, 0)),
                  full((HD, HD)), full((1, HD)), full((1, HD)), full((1, HD)),
                  full((1, HD)), full((1, HD)), full((HD, FF)), full((1, FF)),
                  full((FF, HD)), full((1, HD))],
        out_specs=rows,
        out_shape=jax.ShapeDtypeStruct((N, HD), _f32),
    )(x, parts, Wo, bo, g1, bn1, g2, bn2, W1, bf1, W2, bf2)


_MESH = plsc.VectorSubcoreMesh(core_axis_name="c", subcore_axis_name="s")


SC2 = 200
NS2 = C // SC2
RSB = SC2 // SUB


@functools.partial(
    pl.kernel, mesh=_MESH,
    out_type=[jax.ShapeDtypeStruct((E, HD), _f32),
              jax.ShapeDtypeStruct((E, HD), _f32),
              jax.ShapeDtypeStruct((E, HD), _f32)],
    scratch_types=[pltpu.VMEM((NSUB, SUB), jnp.int32),
                   pltpu.VMEM((NSUB, SUB), jnp.int32),
                   pltpu.VMEM((SC2, HD), _f32),
                   pltpu.VMEM((SC2, HD), _f32),
                   pltpu.VMEM((SC2, HD), _f32),
                   pltpu.SemaphoreType.DMA],
)
def _sc_gather_qkv(src_hbm, dst_hbm, q_hbm, k_hbm, v_hbm,
                   qg_hbm, kg_hbm, vg_hbm, srcv, dstv, qbuf, kbuf, vbuf, sem):
    wid = lax.axis_index("s") * NC + lax.axis_index("c")
    ebase = wid * EPT

    def chunk(i, _):
        base = pl.multiple_of(ebase + i * C, C)
        row0 = pl.multiple_of(base // SUB, NSUB)
        pltpu.sync_copy(src_hbm.at[pl.ds(row0, NSUB)], srcv)
        pltpu.sync_copy(dst_hbm.at[pl.ds(row0, NSUB)], dstv)
        for s in range(NS2):
            cps = []
            for b in range(RSB):
                r = s * RSB + b
                cps.append(pltpu.async_copy(
                    q_hbm.at[srcv.at[r]], qbuf.at[pl.ds(b * SUB, SUB)], sem))
                cps.append(pltpu.async_copy(
                    k_hbm.at[dstv.at[r]], kbuf.at[pl.ds(b * SUB, SUB)], sem))
                cps.append(pltpu.async_copy(
                    v_hbm.at[dstv.at[r]], vbuf.at[pl.ds(b * SUB, SUB)], sem))
            for cp in cps:
                cp.wait()
            sbase = pl.multiple_of(base + s * SC2, SC2)
            pltpu.sync_copy(qbuf, qg_hbm.at[pl.ds(sbase, SC2)])
            pltpu.sync_copy(kbuf, kg_hbm.at[pl.ds(sbase, SC2)])
            pltpu.sync_copy(vbuf, vg_hbm.at[pl.ds(sbase, SC2)])
        return 0

    lax.fori_loop(0, NCH, chunk, 0)


@functools.partial(
    pl.kernel, mesh=_MESH,
    out_type=jax.ShapeDtypeStruct((NC, NPAD, HD), _f32),
    scratch_types=[pltpu.VMEM((NSUB, SUB), jnp.int32),
                   pltpu.VMEM((SC2, HD), _f32),
                   pltpu.VMEM_SHARED((NPAD, HD), _f32),
                   pltpu.SemaphoreType.DMA],
)
def _sc_aggregate(src_hbm, wv_hbm, out_hbm, srcv, vbuf, acc, sem):
    cid = lax.axis_index("c")
    sid = lax.axis_index("s")
    wid = sid * NC + cid
    ebase = wid * EPT

    def zw(r, _):
        for cc in range(HD // 16):
            vbuf[r, pl.ds(cc * 16, 16)] = jnp.zeros((16,), _f32)
        return 0

    lax.fori_loop(0, SC2, zw, 0)
    prow0 = pl.multiple_of(sid * RPS, RPS)
    for j in range(RPS // SC2):
        pltpu.sync_copy(vbuf, acc.at[pl.ds(prow0 + j * SC2, SC2)])
    rem = RPS - (RPS // SC2) * SC2
    if rem:
        pltpu.sync_copy(vbuf.at[pl.ds(0, rem)],
                        acc.at[pl.ds(prow0 + RPS - rem, rem)])
    plsc.subcore_barrier()

    def chunk(i, _):
        base = pl.multiple_of(ebase + i * C, C)
        row0 = pl.multiple_of(base // SUB, NSUB)
        pltpu.sync_copy(src_hbm.at[pl.ds(row0, NSUB)], srcv)
        for s in range(NS2):
            sbase = pl.multiple_of(base + s * SC2, SC2)
            pltpu.sync_copy(wv_hbm.at[pl.ds(sbase, SC2)], vbuf)
            for b in range(RSB):
                pltpu.sync_copy(vbuf.at[pl.ds(b * SUB, SUB)],
                                acc.at[srcv.at[s * RSB + b]], add=True)
        return 0

    lax.fori_loop(0, NCH, chunk, 0)
    plsc.subcore_barrier()
    pltpu.sync_copy(acc.at[pl.ds(prow0, RPS)],
                    out_hbm.at[cid, pl.ds(prow0, RPS)])


def kernel(x, edge_index, edge_attr, Wq, bq, Wk, bk, Wv, bv, We, be, Wo, bo,
           g1, bn1, g2, bn2, W1, bf1, W2, bf2):
    src2d = edge_index[0].reshape(E // SUB, SUB)
    dst2d = edge_index[1].reshape(E // SUB, SUB)
    q, k, v = _qkv_call(x, Wq, bq.reshape(1, HD), Wk, bk.reshape(1, HD),
                        Wv, bv.reshape(1, HD))
    qg, kg, vg = _sc_gather_qkv(src2d, dst2d, q, k, v)
    s, m, z = _score_call(qg, kg, edge_attr, We, be.reshape(1, H))
    wv = _wv_call(s, m, z, vg)
    parts = _sc_aggregate(src2d, wv)
    return _post_call(x, parts, Wo, bo.reshape(1, HD), g1.reshape(1, HD),
                      bn1.reshape(1, HD), g2.reshape(1, HD), bn2.reshape(1, HD),
                      W1, bf1.reshape(1, FF), W2, bf2.reshape(1, HD))
